# Initial kernel scaffold; baseline (speedup 1.0000x reference)
import jax
import jax.numpy as jnp
from jax.experimental import pallas as pl


def kernel(x_user, x_movie, W1_um_r, W1_um_root, b1_m, W1_mu_r, W1_mu_root, b1_u, W2_um_r, W2_um_root, b2_m, W2_mu_r, W2_mu_root, b2_u, edge_index_user_movie, edge_index_movie_user, edge_label_index):
    raise NotImplementedError("write your pallas kernel here")


# speedup vs baseline: 4.8360x; 4.8360x over previous
import functools
import jax
import jax.numpy as jnp
from jax import lax
from jax.experimental import pallas as pl
from jax.experimental.pallas import tpu as pltpu
from jax.experimental.pallas import tpu_sc as plsc

N = 10000
D = 128
E = 320000
L = 100000

NC, NS = 2, 16
NW = NC * NS
C = 128
EPT = 10112
NCHUNK = EPT // C
NPAD = 10240
RPS = NPAD // NS
LPT = 3200
LCHUNK = LPT // C

_mesh = plsc.VectorSubcoreMesh(core_axis_name="c", subcore_axis_name="s")


def _pad_tiles(arr, per_tile, pad_per_tile, pad_vals):
    a = arr.reshape(NW, per_tile)
    pad = jnp.broadcast_to(pad_vals[None, :], (NW, pad_per_tile))
    return jnp.concatenate([a, pad], axis=1).reshape(-1)


def _agg_body(with_count, xt, srcA, dstA, *rest):
    if with_count:
        acc_out, cnt_out = rest[0], rest[1]
        scr = rest[2:]
    else:
        acc_out = rest[0]
        cnt_out = None
        scr = rest[1:]
    srcv, dstv, rows, zbuf, ones, zcnt, accS, cntS, sem = scr

    cid = lax.axis_index("c")
    sid = lax.axis_index("s")

    zero16 = jnp.zeros((16,), jnp.float32)
    for i in range(64):
        for j in range(D // 16):
            zbuf[i, pl.ds(j * 16, 16)] = zero16
    for j in range(RPS // 16):
        zcnt[pl.ds(j * 16, 16)] = zero16
    if with_count:
        one16 = jnp.ones((16,), jnp.float32)
        for j in range(C // 16):
            ones[pl.ds(j * 16, 16)] = one16

    for t in range(RPS // 64):
        pltpu.sync_copy(zbuf, accS.at[pl.ds(sid * RPS + t * 64, 64)])
    pltpu.sync_copy(zcnt, cntS.at[pl.ds(sid * RPS, RPS)])
    plsc.subcore_barrier()

    ebase = (cid * NS + sid) * EPT

    def chunk(k, _):
        b = pl.multiple_of(ebase + k * C, 8)
        pltpu.sync_copy(srcA.at[pl.ds(b, C)], srcv)
        pltpu.sync_copy(dstA.at[pl.ds(b, C)], dstv)
        pltpu.async_copy(xt.at[srcv], rows, sem).wait()
        pltpu.sync_copy(rows, accS.at[dstv], add=True)
        if with_count:
            pltpu.sync_copy(ones, cntS.at[dstv], add=True)
        return 0

    lax.fori_loop(0, NCHUNK, chunk, 0)
    plsc.subcore_barrier()

    r0 = sid * RPS
    pltpu.sync_copy(accS.at[pl.ds(r0, RPS)], acc_out.at[cid, pl.ds(r0, RPS)])
    if with_count:
        pltpu.sync_copy(cntS.at[pl.ds(r0, RPS)],
                        cnt_out.at[cid, pl.ds(r0, RPS)])


def _make_agg(with_count):
    outs = [jax.ShapeDtypeStruct((NC, NPAD, D), jnp.float32)]
    if with_count:
        outs.append(jax.ShapeDtypeStruct((NC, NPAD), jnp.float32))
    scratch = [
        pltpu.VMEM((C,), jnp.int32),
        pltpu.VMEM((C,), jnp.int32),
        pltpu.VMEM((C, D), jnp.float32),
        pltpu.VMEM((64, D), jnp.float32),
        pltpu.VMEM((C,), jnp.float32),
        pltpu.VMEM((RPS,), jnp.float32),
        pltpu.VMEM_SHARED((NPAD, D), jnp.float32),
        pltpu.VMEM_SHARED((NPAD,), jnp.float32),
        pltpu.SemaphoreType.DMA,
    ]
    return pl.kernel(
        functools.partial(_agg_body, with_count),
        out_type=tuple(outs) if with_count else outs[0],
        mesh=_mesh,
        scratch_types=scratch,
    )


_agg_cnt = _make_agg(True)
_agg = _make_agg(False)


def _dec_body(zu, zm, srcA, dstA, out, idxs, idxd, rs, rd, obuf, sem):
    cid = lax.axis_index("c")
    sid = lax.axis_index("s")
    base = (cid * NS + sid) * LPT

    def chunk(k, _):
        b = pl.multiple_of(base + k * C, 8)
        pltpu.sync_copy(srcA.at[pl.ds(b, C)], idxs)
        pltpu.sync_copy(dstA.at[pl.ds(b, C)], idxd)
        pltpu.async_copy(zu.at[idxs], rs, sem).wait()
        pltpu.async_copy(zm.at[idxd], rd, sem).wait()

        lane = lax.iota(jnp.int32, 16)

        def grp(g, _):
            d = jnp.zeros((16,), jnp.float32)
            for q in range(16):
                p = g * 16 + q
                a = rs[p, pl.ds(0, 16)] * rd[p, pl.ds(0, 16)]
                for j in range(1, D // 16):
                    a = a + rs[p, pl.ds(j * 16, 16)] * rd[p, pl.ds(j * 16, 16)]
                for sh in (8, 4, 2, 1):
                    a = a + a[lane ^ sh]
                d = jnp.where(lane == q, a, d)
            obuf[pl.ds(g * 16, 16)] = d
            return 0

        lax.fori_loop(0, C // 16, grp, 0)
        pltpu.sync_copy(obuf, out.at[pl.ds(b, C)])
        return 0

    lax.fori_loop(0, LCHUNK, chunk, 0)


_decoder = pl.kernel(
    _dec_body,
    out_type=jax.ShapeDtypeStruct((NW * LPT,), jnp.float32),
    mesh=_mesh,
    scratch_types=[
        pltpu.VMEM((C,), jnp.int32),
        pltpu.VMEM((C,), jnp.int32),
        pltpu.VMEM((C, D), jnp.float32),
        pltpu.VMEM((C, D), jnp.float32),
        pltpu.VMEM((C,), jnp.float32),
        pltpu.SemaphoreType.DMA,
    ],
)


_RB = 1024
_GRID = NPAD // _RB


def _mm_body(x, w, o):
    o[...] = jnp.dot(x[...], w[...], preferred_element_type=jnp.float32)


def _mm(x, w):
    return pl.pallas_call(
        _mm_body,
        grid=(_GRID,),
        in_specs=[
            pl.BlockSpec((_RB, D), lambda i: (i, 0)),
            pl.BlockSpec((D, D), lambda i: (0, 0)),
        ],
        out_specs=pl.BlockSpec((_RB, D), lambda i: (i, 0)),
        out_shape=jax.ShapeDtypeStruct((NPAD, D), jnp.float32),
    )(x, w)


def _fin1_body(pacc, pcnt, x, wroot, wnext, b, h, t):
    acc = pacc[0] + pacc[1]
    cnt = pcnt[0] + pcnt[1]
    inv = 1.0 / jnp.maximum(cnt, 1.0)
    hv = jnp.maximum(
        acc * inv[:, None]
        + jnp.dot(x[...], wroot[...], preferred_element_type=jnp.float32)
        + b[...], 0.0)
    h[...] = hv
    t[...] = jnp.dot(hv, wnext[...], preferred_element_type=jnp.float32)


def _finish1(pacc, pcnt, x, wroot, b, wnext):
    return pl.pallas_call(
        _fin1_body,
        grid=(_GRID,),
        in_specs=[
            pl.BlockSpec((NC, _RB, D), lambda i: (0, i, 0)),
            pl.BlockSpec((NC, _RB), lambda i: (0, i)),
            pl.BlockSpec((_RB, D), lambda i: (i, 0)),
            pl.BlockSpec((D, D), lambda i: (0, 0)),
            pl.BlockSpec((D, D), lambda i: (0, 0)),
            pl.BlockSpec((1, D), lambda i: (0, 0)),
        ],
        out_specs=[
            pl.BlockSpec((_RB, D), lambda i: (i, 0)),
            pl.BlockSpec((_RB, D), lambda i: (i, 0)),
        ],
        out_shape=[
            jax.ShapeDtypeStruct((NPAD, D), jnp.float32),
            jax.ShapeDtypeStruct((NPAD, D), jnp.float32),
        ],
    )(pacc, pcnt, x, wroot, wnext, b.reshape(1, D))


def _fin2_body(pacc, pcnt, x, wroot, b, z):
    acc = pacc[0] + pacc[1]
    cnt = pcnt[0] + pcnt[1]
    inv = 1.0 / jnp.maximum(cnt, 1.0)
    z[...] = (acc * inv[:, None]
              + jnp.dot(x[...], wroot[...], preferred_element_type=jnp.float32)
              + b[...])


def _finish2(pacc, pcnt, x, wroot, b):
    return pl.pallas_call(
        _fin2_body,
        grid=(_GRID,),
        in_specs=[
            pl.BlockSpec((NC, _RB, D), lambda i: (0, i, 0)),
            pl.BlockSpec((NC, _RB), lambda i: (0, i)),
            pl.BlockSpec((_RB, D), lambda i: (i, 0)),
            pl.BlockSpec((D, D), lambda i: (0, 0)),
            pl.BlockSpec((1, D), lambda i: (0, 0)),
        ],
        out_specs=pl.BlockSpec((_RB, D), lambda i: (i, 0)),
        out_shape=jax.ShapeDtypeStruct((NPAD, D), jnp.float32),
    )(pacc, pcnt, x, wroot, b.reshape(1, D))


def kernel(x_user, x_movie,
           W1_um_r, W1_um_root, b1_m, W1_mu_r, W1_mu_root, b1_u,
           W2_um_r, W2_um_root, b2_m, W2_mu_r, W2_mu_root, b2_u,
           edge_index_user_movie, edge_index_movie_user, edge_label_index):
    eum = edge_index_user_movie.astype(jnp.int32)
    emu = edge_index_movie_user.astype(jnp.int32)
    eli = edge_label_index.astype(jnp.int32)
    xp_user = jnp.pad(x_user, ((0, NPAD - N), (0, 0)))
    xp_movie = jnp.pad(x_movie, ((0, NPAD - N), (0, 0)))

    epad = EPT - E // NW
    pad_src = (jnp.arange(epad, dtype=jnp.int32) % 16)
    pad_dst = N + (jnp.arange(epad, dtype=jnp.int32) % (NPAD - N))
    src_um = _pad_tiles(eum[0], E // NW, epad, pad_src)
    dst_um = _pad_tiles(eum[1], E // NW, epad, pad_dst)
    src_mu = _pad_tiles(emu[0], E // NW, epad, pad_src)
    dst_mu = _pad_tiles(emu[1], E // NW, epad, pad_dst)

    lpad = LPT - L // NW
    pad_l = (jnp.arange(lpad, dtype=jnp.int32) % 16)
    src_l = _pad_tiles(eli[0], L // NW, lpad, pad_l)
    dst_l = _pad_tiles(eli[1], L // NW, lpad, pad_l)

    t_u1 = _mm(xp_user, W1_um_r)
    t_m1 = _mm(xp_movie, W1_mu_r)
    pacc_m, pcnt_m = _agg_cnt(t_u1, src_um, dst_um)
    pacc_u, pcnt_u = _agg_cnt(t_m1, src_mu, dst_mu)
    h_m, t_m2 = _finish1(pacc_m, pcnt_m, xp_movie, W1_um_root, b1_m, W2_mu_r)
    h_u, t_u2 = _finish1(pacc_u, pcnt_u, xp_user, W1_mu_root, b1_u, W2_um_r)

    pacc_m2 = _agg(t_u2, src_um, dst_um)
    pacc_u2 = _agg(t_m2, src_mu, dst_mu)
    z_m = _finish2(pacc_m2, pcnt_m, h_m, W2_um_root, b2_m)
    z_u = _finish2(pacc_u2, pcnt_u, h_u, W2_mu_root, b2_u)

    out_pad = _decoder(z_u, z_m, src_l, dst_l)
    return out_pad.reshape(NW, LPT)[:, :L // NW].reshape(-1)
